# baseline (speedup 1.0000x reference)
import functools

import jax
import jax.numpy as jnp
from jax import lax
from jax.experimental import pallas as pl
from jax.experimental.pallas import tpu as pltpu
from jax.experimental.pallas import tpu_sc as plsc

STATE_DIM = 64
TOTAL = 4 * 2048 * 2048
NC, NS = 2, 16
NW = NC * NS
REGION = TOTAL // NW
SLICE = 16384
NSLICES = REGION // SLICE
CHUNKS = SLICE // STATE_DIM
UNROLL = 4

_mesh = plsc.VectorSubcoreMesh(core_axis_name="c", subcore_axis_name="s")


@functools.partial(
    pl.kernel,
    mesh=_mesh,
    out_type=jax.ShapeDtypeStruct((TOTAL,), jnp.float32),
    compiler_params=pltpu.CompilerParams(needs_layout_passes=False),
    scratch_types=[
        pltpu.VMEM((SLICE,), jnp.float32),
        pltpu.VMEM((SLICE,), jnp.float32),
        pltpu.VMEM((SLICE,), jnp.float32),
        pltpu.VMEM((SLICE,), jnp.float32),
        pltpu.VMEM((STATE_DIM,), jnp.int32),
        pltpu.SemaphoreType.DMA,
        pltpu.SemaphoreType.DMA,
        pltpu.SemaphoreType.DMA,
        pltpu.SemaphoreType.DMA,
    ],
)
def _permute_sc(x_hbm, idx_hbm, o_hbm, in0, in1, out0, out1, perm_v,
                si0, si1, so0, so1):
    wid = lax.axis_index("s") * NC + lax.axis_index("c")
    base = wid * REGION

    pltpu.sync_copy(idx_hbm, perm_v)
    pq = [perm_v[pl.ds(q * 16, 16)] for q in range(4)]

    inb = [in0, in1]
    outb = [out0, out1]
    sin = [si0, si1]
    sout = [so0, so1]

    pltpu.async_copy(x_hbm.at[pl.ds(base, SLICE)], in0, si0)
    pltpu.async_copy(x_hbm.at[pl.ds(base + SLICE, SLICE)], in1, si1)

    def outer(i2, carry):
        for b in range(2):
            i = i2 * 2 + b
            off = base + i * SLICE
            pltpu.make_async_copy(
                x_hbm.at[pl.ds(off, SLICE)], inb[b], sin[b]).wait()

            @pl.when(i2 > 0)
            def _wait_out():
                pltpu.make_async_copy(
                    outb[b], o_hbm.at[pl.ds(off, SLICE)], sout[b]).wait()

            def chunk(g, c2):
                for u in range(UNROLL):
                    cb = (g * UNROLL + u) * STATE_DIM
                    for q in range(4):
                        vals = plsc.load_gather(inb[b], [pq[q] + cb])
                        outb[b][pl.ds(cb + q * 16, 16)] = vals
                return c2

            lax.fori_loop(0, CHUNKS // UNROLL, chunk, 0)

            pltpu.async_copy(outb[b], o_hbm.at[pl.ds(off, SLICE)], sout[b])

            @pl.when(i2 < NSLICES // 2 - 1)
            def _prefetch():
                pltpu.async_copy(
                    x_hbm.at[pl.ds(off + 2 * SLICE, SLICE)], inb[b], sin[b])
        return carry

    lax.fori_loop(0, NSLICES // 2, outer, 0)

    for b in range(2):
        off = base + (NSLICES - 2 + b) * SLICE
        pltpu.make_async_copy(
            outb[b], o_hbm.at[pl.ds(off, SLICE)], sout[b]).wait()


def kernel(x, index_1):
    out_flat = _permute_sc(x.reshape(-1), index_1)
    return out_flat.reshape(x.shape)
